# baseline (speedup 1.0000x reference)
import jax
import jax.numpy as jnp
from jax import lax
from jax.experimental import pallas as pl
from jax.experimental.pallas import tpu as pltpu
from jax.experimental.pallas import tpu_sc as plsc

NUM_CLASSES = 16
TIMESTEPS = 300
N_TOKENS = 131072

NUM_CORES = 2
NUM_SUBCORES = 16
LANES = 16
NUM_WORKERS = NUM_CORES * NUM_SUBCORES
TOK_PER_WORKER = N_TOKENS // NUM_WORKERS
CHUNK = 2048
NUM_CHUNKS = TOK_PER_WORKER // CHUNK


def _sc_body(x0_hbm, t_hbm, qt_hbm, out_hbm, x_v, t_v, qt_v):
    wid = lax.axis_index("s") * NUM_CORES + lax.axis_index("c")
    base = wid * TOK_PER_WORKER

    pltpu.sync_copy(qt_hbm, qt_v)
    pltpu.sync_copy(t_hbm.at[pl.ds(base, TOK_PER_WORKER)], t_v)

    zeros = jnp.zeros((LANES,), jnp.int32)
    ones = jnp.ones((LANES,), jnp.int32)

    def chunk_body(c, carry):
        pltpu.sync_copy(x0_hbm.at[pl.ds(base + c * CHUNK, CHUNK), :], x_v)

        def block(i, carry2):
            tok = i * LANES + lax.iota(jnp.int32, LANES)
            tvec = t_v[pl.ds(c * CHUNK + i * LANES, LANES)]
            d = plsc.load_gather(qt_v, [tvec, zeros, zeros])
            o = plsc.load_gather(qt_v, [tvec, zeros, ones])
            w = d - o
            cs = [
                plsc.load_gather(x_v, [tok, jnp.full((LANES,), j, jnp.int32)])
                for j in range(NUM_CLASSES)
            ]
            s = cs[0]
            for j in range(1, NUM_CLASSES):
                s = s + cs[j]
            os = o * s
            for j in range(NUM_CLASSES):
                plsc.store_scatter(
                    x_v, [tok, jnp.full((LANES,), j, jnp.int32)], w * cs[j] + os
                )
            return carry2

        lax.fori_loop(0, CHUNK // LANES, block, 0)
        pltpu.sync_copy(x_v, out_hbm.at[pl.ds(base + c * CHUNK, CHUNK), :])
        return carry

    lax.fori_loop(0, NUM_CHUNKS, chunk_body, 0)


@jax.jit
def _run(x0, t, Qt_bar):
    mesh = plsc.VectorSubcoreMesh(core_axis_name="c", subcore_axis_name="s")
    return pl.kernel(
        _sc_body,
        out_type=jax.ShapeDtypeStruct((N_TOKENS, NUM_CLASSES), jnp.float32),
        mesh=mesh,
        scratch_types=[
            pltpu.VMEM((CHUNK, NUM_CLASSES), jnp.float32),
            pltpu.VMEM((TOK_PER_WORKER,), jnp.int32),
            pltpu.VMEM((TIMESTEPS, NUM_CLASSES, NUM_CLASSES), jnp.float32),
        ],
        compiler_params=pltpu.CompilerParams(
            needs_layout_passes=False, use_tc_tiling_on_sc=False
        ),
    )(x0, t, Qt_bar)


def kernel(x0, t, Qt_bar):
    return _run(x0, t.astype(jnp.int32), Qt_bar)


# speedup vs baseline: 3.5304x; 3.5304x over previous
import jax
import jax.numpy as jnp
from jax import lax
from jax.experimental import pallas as pl
from jax.experimental.pallas import tpu as pltpu
from jax.experimental.pallas import tpu_sc as plsc

NUM_CLASSES = 16
TIMESTEPS = 300
N_TOKENS = 131072

NUM_CORES = 2
NUM_SUBCORES = 16
LANES = 16
NUM_WORKERS = NUM_CORES * NUM_SUBCORES
TOK_PER_WORKER = N_TOKENS // NUM_WORKERS
CHUNK = 2048
NUM_CHUNKS = TOK_PER_WORKER // CHUNK
QT_FLAT = TIMESTEPS * NUM_CLASSES * NUM_CLASSES


def _sc_body(x0_hbm, t_hbm, qt_hbm, out_hbm, x_v, t_v, qt_v):
    wid = lax.axis_index("s") * NUM_CORES + lax.axis_index("c")
    base = wid * TOK_PER_WORKER

    pltpu.sync_copy(qt_hbm, qt_v)
    pltpu.sync_copy(t_hbm.at[pl.ds(base, TOK_PER_WORKER)], t_v)

    def chunk_body(c, carry):
        pltpu.sync_copy(x0_hbm.at[:, pl.ds(base + c * CHUNK, CHUNK)], x_v)

        def block(g, carry2):
            tvec = t_v[pl.ds(c * CHUNK + g * LANES, LANES)]
            toff = tvec * (NUM_CLASSES * NUM_CLASSES)
            d = plsc.load_gather(qt_v, [toff])
            o = plsc.load_gather(qt_v, [toff + 1])
            w = d - o
            rs = [x_v[j, pl.ds(g * LANES, LANES)] for j in range(NUM_CLASSES)]
            s = rs[0]
            for j in range(1, NUM_CLASSES):
                s = s + rs[j]
            os = o * s
            for j in range(NUM_CLASSES):
                x_v[j, pl.ds(g * LANES, LANES)] = w * rs[j] + os
            return carry2

        lax.fori_loop(0, CHUNK // LANES, block, 0)
        pltpu.sync_copy(x_v, out_hbm.at[:, pl.ds(base + c * CHUNK, CHUNK)])
        return carry

    lax.fori_loop(0, NUM_CHUNKS, chunk_body, 0)


@jax.jit
def _run(x0t, t, qt_flat):
    mesh = plsc.VectorSubcoreMesh(core_axis_name="c", subcore_axis_name="s")
    return pl.kernel(
        _sc_body,
        out_type=jax.ShapeDtypeStruct((NUM_CLASSES, N_TOKENS), jnp.float32),
        mesh=mesh,
        scratch_types=[
            pltpu.VMEM((NUM_CLASSES, CHUNK), jnp.float32),
            pltpu.VMEM((TOK_PER_WORKER,), jnp.int32),
            pltpu.VMEM((QT_FLAT,), jnp.float32),
        ],
        compiler_params=pltpu.CompilerParams(needs_layout_passes=False),
    )(x0t, t, qt_flat)


def kernel(x0, t, Qt_bar):
    out_t = _run(x0.T, t.astype(jnp.int32), Qt_bar.reshape(QT_FLAT))
    return out_t.T


# speedup vs baseline: 5.0624x; 1.4340x over previous
import jax
import jax.numpy as jnp
from jax import lax
from jax.experimental import pallas as pl
from jax.experimental.pallas import tpu as pltpu
from jax.experimental.pallas import tpu_sc as plsc

NUM_CLASSES = 16
TIMESTEPS = 300
N_TOKENS = 131072

NUM_CORES = 2
NUM_SUBCORES = 16
LANES = 16
NUM_WORKERS = NUM_CORES * NUM_SUBCORES
TOK_PER_WORKER = N_TOKENS // NUM_WORKERS
CHUNK = 2048
MAT = NUM_CLASSES * NUM_CLASSES

EX_ROWS = 32
EX_TILES = 10
T_PAD = EX_ROWS * EX_TILES
QT_PAD_FLAT = T_PAD * MAT


def _sc_body(x0_hbm, t_hbm, qt_hbm, out_hbm,
             xa_v, xb_v, t_v, stage_v, dotab_v, do_shared,
             sem_ia, sem_ib, sem_oa, sem_ob, sem_t):
    cid = lax.axis_index("c")
    sid = lax.axis_index("s")
    wid = sid * NUM_CORES + cid
    base = wid * TOK_PER_WORKER

    in_a = pltpu.async_copy(x0_hbm.at[:, pl.ds(base, CHUNK)], xa_v, sem_ia)
    in_b = pltpu.async_copy(x0_hbm.at[:, pl.ds(base + CHUNK, CHUNK)], xb_v, sem_ib)
    t_cp = pltpu.async_copy(t_hbm.at[pl.ds(base, TOK_PER_WORKER)], t_v, sem_t)

    @pl.when(sid < EX_TILES)
    def _extract():
        row0 = sid * EX_ROWS
        pltpu.sync_copy(qt_hbm.at[pl.ds(row0 * MAT, EX_ROWS * MAT)], stage_v)
        for b in range(EX_ROWS // LANES):
            rloc = b * LANES + lax.iota(jnp.int32, LANES)
            d = plsc.load_gather(stage_v, [rloc * MAT])
            o = plsc.load_gather(stage_v, [rloc * MAT + 1])
            dotab_v[pl.ds(row0 + b * LANES, LANES)] = d
            dotab_v[pl.ds(T_PAD + row0 + b * LANES, LANES)] = o
        pltpu.sync_copy(dotab_v.at[pl.ds(row0, EX_ROWS)],
                        do_shared.at[pl.ds(row0, EX_ROWS)])
        pltpu.sync_copy(dotab_v.at[pl.ds(T_PAD + row0, EX_ROWS)],
                        do_shared.at[pl.ds(T_PAD + row0, EX_ROWS)])

    plsc.subcore_barrier()
    pltpu.sync_copy(do_shared, dotab_v)
    t_cp.wait()

    def _compute(c, x_v):
        def block(g, carry):
            tvec = t_v[pl.ds(c * CHUNK + g * LANES, LANES)]
            d = plsc.load_gather(dotab_v, [tvec])
            o = plsc.load_gather(dotab_v, [tvec + T_PAD])
            w = d - o
            rs = [x_v[j, pl.ds(g * LANES, LANES)] for j in range(NUM_CLASSES)]
            s = rs[0]
            for j in range(1, NUM_CLASSES):
                s = s + rs[j]
            os = o * s
            for j in range(NUM_CLASSES):
                x_v[j, pl.ds(g * LANES, LANES)] = w * rs[j] + os
            return carry

        lax.fori_loop(0, CHUNK // LANES, block, 0)

    in_a.wait()
    _compute(0, xa_v)
    out_a = pltpu.async_copy(xa_v, out_hbm.at[:, pl.ds(base, CHUNK)], sem_oa)
    in_b.wait()
    _compute(1, xb_v)
    out_b = pltpu.async_copy(xb_v, out_hbm.at[:, pl.ds(base + CHUNK, CHUNK)], sem_ob)
    out_a.wait()
    out_b.wait()


@jax.jit
def _run(x0t, t, qt_flat):
    mesh = plsc.VectorSubcoreMesh(core_axis_name="c", subcore_axis_name="s")
    return pl.kernel(
        _sc_body,
        out_type=jax.ShapeDtypeStruct((NUM_CLASSES, N_TOKENS), jnp.float32),
        mesh=mesh,
        scratch_types=[
            pltpu.VMEM((NUM_CLASSES, CHUNK), jnp.float32),
            pltpu.VMEM((NUM_CLASSES, CHUNK), jnp.float32),
            pltpu.VMEM((TOK_PER_WORKER,), jnp.int32),
            pltpu.VMEM((EX_ROWS * MAT,), jnp.float32),
            pltpu.VMEM((2 * T_PAD,), jnp.float32),
            pltpu.VMEM_SHARED((2 * T_PAD,), jnp.float32),
            pltpu.SemaphoreType.DMA,
            pltpu.SemaphoreType.DMA,
            pltpu.SemaphoreType.DMA,
            pltpu.SemaphoreType.DMA,
            pltpu.SemaphoreType.DMA,
        ],
        compiler_params=pltpu.CompilerParams(needs_layout_passes=False),
    )(x0t, t, qt_flat)


def kernel(x0, t, Qt_bar):
    qt_flat = jnp.pad(Qt_bar.reshape(TIMESTEPS * MAT),
                      (0, (T_PAD - TIMESTEPS) * MAT))
    out_t = _run(x0.T, t.astype(jnp.int32), qt_flat)
    return out_t.T
